# baseline (speedup 1.0000x reference)
import functools

import jax
import jax.numpy as jnp
from jax import lax
from jax.experimental import pallas as pl
from jax.experimental.pallas import tpu as pltpu

N1, N2, N4, N8, N16 = 500000, 200000, 80000, 30000, 10000
BR = 2000


def _lrelu(x):
    return jnp.where(x >= 0, x, 0.1 * x)


def _bdot(a, b):
    return jnp.dot(a.astype(jnp.bfloat16), b.astype(jnp.bfloat16),
                   preferred_element_type=jnp.float32)


def _mm_stats_body(x_ref, w_ref, b_ref, h_ref, st_ref):
    h = _lrelu(_bdot(x_ref[...], w_ref[...]) + b_ref[...])
    h_ref[...] = h

    @pl.when(pl.program_id(0) == 0)
    def _():
        st_ref[...] = jnp.zeros_like(st_ref)

    st_ref[...] += jnp.stack([jnp.sum(h, axis=0), jnp.sum(h * h, axis=0)])


def _mm_stats(x, w, b):
    n, cin = x.shape
    cout = w.shape[1]
    grid = n // BR
    return pl.pallas_call(
        _mm_stats_body,
        grid=(grid,),
        in_specs=[
            pl.BlockSpec((BR, cin), lambda i: (i, 0)),
            pl.BlockSpec((cin, cout), lambda i: (0, 0)),
            pl.BlockSpec((1, cout), lambda i: (0, 0)),
        ],
        out_specs=[
            pl.BlockSpec((BR, cout), lambda i: (i, 0)),
            pl.BlockSpec((2, cout), lambda i: (0, 0)),
        ],
        out_shape=[
            jax.ShapeDtypeStruct((n, cout), jnp.float32),
            jax.ShapeDtypeStruct((2, cout), jnp.float32),
        ],
    )(x, w, b)


def _mm_affine_body(x_ref, s_ref, t_ref, w_ref, b_ref, o_ref):
    hb = x_ref[...] * s_ref[...] + t_ref[...]
    o_ref[...] = _lrelu(_bdot(hb, w_ref[...]) + b_ref[...])


def _mm_affine(x, s, t, w, b):
    n, cin = x.shape
    cout = w.shape[1]
    return pl.pallas_call(
        _mm_affine_body,
        grid=(n // BR,),
        in_specs=[
            pl.BlockSpec((BR, cin), lambda i: (i, 0)),
            pl.BlockSpec((1, cin), lambda i: (0, 0)),
            pl.BlockSpec((1, cin), lambda i: (0, 0)),
            pl.BlockSpec((cin, cout), lambda i: (0, 0)),
            pl.BlockSpec((1, cout), lambda i: (0, 0)),
        ],
        out_specs=pl.BlockSpec((BR, cout), lambda i: (i, 0)),
        out_shape=jax.ShapeDtypeStruct((n, cout), jnp.float32),
    )(x, s, t, w, b)


def _dec_body(cur_ref, last_ref, mw_ref, mb_ref, dw1_ref, dw2_ref, db_ref,
              f_ref, st_ref):
    skip = _lrelu(_bdot(cur_ref[...], mw_ref[...]) + mb_ref[...])
    f = (_bdot(skip, dw1_ref[...]) + _bdot(last_ref[...], dw2_ref[...])
         + db_ref[...])
    f_ref[...] = f

    @pl.when(pl.program_id(0) == 0)
    def _():
        st_ref[...] = jnp.zeros_like(st_ref)

    st_ref[...] += jnp.stack([jnp.sum(f, axis=0), jnp.sum(f * f, axis=0)])


def _dec_mm(cur, last, mw, mb, dw1, dw2, db):
    n = cur.shape[0]
    c = 64
    return pl.pallas_call(
        _dec_body,
        grid=(n // BR,),
        in_specs=[
            pl.BlockSpec((BR, c), lambda i: (i, 0)),
            pl.BlockSpec((BR, c), lambda i: (i, 0)),
            pl.BlockSpec((c, c), lambda i: (0, 0)),
            pl.BlockSpec((1, c), lambda i: (0, 0)),
            pl.BlockSpec((c, c), lambda i: (0, 0)),
            pl.BlockSpec((c, c), lambda i: (0, 0)),
            pl.BlockSpec((1, c), lambda i: (0, 0)),
        ],
        out_specs=[
            pl.BlockSpec((BR, c), lambda i: (i, 0)),
            pl.BlockSpec((2, c), lambda i: (0, 0)),
        ],
        out_shape=[
            jax.ShapeDtypeStruct((n, c), jnp.float32),
            jax.ShapeDtypeStruct((2, c), jnp.float32),
        ],
    )(cur, last, mw, mb, dw1, dw2, db)


def _affine_body(f_ref, s_ref, t_ref, o_ref):
    o_ref[...] = _lrelu(f_ref[...] * s_ref[...] + t_ref[...])


def _affine(f, s, t):
    n, c = f.shape
    return pl.pallas_call(
        _affine_body,
        grid=(n // BR,),
        in_specs=[
            pl.BlockSpec((BR, c), lambda i: (i, 0)),
            pl.BlockSpec((1, c), lambda i: (0, 0)),
            pl.BlockSpec((1, c), lambda i: (0, 0)),
        ],
        out_specs=pl.BlockSpec((BR, c), lambda i: (i, 0)),
        out_shape=jax.ShapeDtypeStruct((n, c), jnp.float32),
    )(f, s, t)


def _bn_scale_shift(st, n, g, be):
    mean = st[0] / n
    var = st[1] / n - mean * mean
    s = g * lax.rsqrt(var + 1e-5)
    t = be - mean * s
    return s, t


def _segmax(x, idx, num):
    o = jax.ops.segment_max(x, idx, num_segments=num)
    return jnp.where(jnp.isfinite(o), o, 0.0)


def _gather(x, idx):
    return x[idx]


def _enc(x, w1, b1, g, be, w2, b2):
    n = x.shape[0]
    h, st = _mm_stats(x, w1, b1.reshape(1, -1))
    s, t = _bn_scale_shift(st, n, g, be)
    return _mm_affine(h, s.reshape(1, -1), t.reshape(1, -1), w2,
                      b2.reshape(1, -1))


def _ur(cur, last, mw, mb, dw, db, g, be):
    n = cur.shape[0]
    f, st = _dec_mm(cur, last, mw, mb.reshape(1, -1), dw[:64], dw[64:],
                    db.reshape(1, -1))
    s, t = _bn_scale_shift(st, n, g, be)
    return _affine(f, s.reshape(1, -1), t.reshape(1, -1))


def kernel(pt_fea, coors_inv_2, coors_inv_4, coors_inv_8, coors_inv_16,
           enc0_W1, encW1, encB1, encG, encBe, encW2, encB2,
           mlpW, mlpB, decW, decB, decG, decBe):
    p1 = _enc(pt_fea, enc0_W1, encB1[0], encG[0], encBe[0], encW2[0], encB2[0])
    o = _segmax(p1, coors_inv_2, N2)
    p2 = _enc(o, encW1[0], encB1[1], encG[1], encBe[1], encW2[1], encB2[1])
    o = _segmax(p2, coors_inv_4, N4)
    p3 = _enc(o, encW1[1], encB1[2], encG[2], encBe[2], encW2[2], encB2[2])
    o = _segmax(p3, coors_inv_8, N8)
    p4 = _enc(o, encW1[2], encB1[3], encG[3], encBe[3], encW2[3], encB2[3])
    o = _segmax(p4, coors_inv_16, N16)
    p5 = _enc(o, encW1[3], encB1[4], encG[4], encBe[4], encW2[4], encB2[4])
    p5f = _ur(p5, p5, mlpW[0], mlpB[0], decW[0], decB[0], decG[0], decBe[0])
    o = _gather(p5f, coors_inv_16)
    p4f = _ur(p4, o, mlpW[1], mlpB[1], decW[1], decB[1], decG[1], decBe[1])
    o = _gather(p4f, coors_inv_8)
    p3f = _ur(p3, o, mlpW[2], mlpB[2], decW[2], decB[2], decG[2], decBe[2])
    o = _gather(p3f, coors_inv_4)
    p2f = _ur(p2, o, mlpW[3], mlpB[3], decW[3], decB[3], decG[3], decBe[3])
    o = _gather(p2f, coors_inv_2)
    p1f = _ur(p1, o, mlpW[4], mlpB[4], decW[4], decB[4], decG[4], decBe[4])
    return (p1f, p2f, p3f, p4f, p5f)


# speedup vs baseline: 1.3544x; 1.3544x over previous
import functools

import jax
import jax.numpy as jnp
from jax import lax
from jax.experimental import pallas as pl
from jax.experimental.pallas import tpu as pltpu
from jax.experimental.pallas import tpu_sc as plsc

N1, N2, N4, N8, N16 = 500000, 200000, 80000, 30000, 10000
BR = 2000


def _lrelu(x):
    return jnp.where(x >= 0, x, 0.1 * x)


def _bdot(a, b):
    return jnp.dot(a.astype(jnp.bfloat16), b.astype(jnp.bfloat16),
                   preferred_element_type=jnp.float32)


def _mm_stats_body(x_ref, w_ref, b_ref, h_ref, st_ref):
    h = _lrelu(_bdot(x_ref[...], w_ref[...]) + b_ref[...])
    h_ref[...] = h

    @pl.when(pl.program_id(0) == 0)
    def _():
        st_ref[...] = jnp.zeros_like(st_ref)

    st_ref[...] += jnp.stack([jnp.sum(h, axis=0), jnp.sum(h * h, axis=0)])


def _mm_stats(x, w, b):
    n, cin = x.shape
    cout = w.shape[1]
    grid = n // BR
    return pl.pallas_call(
        _mm_stats_body,
        grid=(grid,),
        in_specs=[
            pl.BlockSpec((BR, cin), lambda i: (i, 0)),
            pl.BlockSpec((cin, cout), lambda i: (0, 0)),
            pl.BlockSpec((1, cout), lambda i: (0, 0)),
        ],
        out_specs=[
            pl.BlockSpec((BR, cout), lambda i: (i, 0)),
            pl.BlockSpec((2, cout), lambda i: (0, 0)),
        ],
        out_shape=[
            jax.ShapeDtypeStruct((n, cout), jnp.float32),
            jax.ShapeDtypeStruct((2, cout), jnp.float32),
        ],
    )(x, w, b)


def _mm_affine_body(x_ref, s_ref, t_ref, w_ref, b_ref, o_ref):
    hb = x_ref[...] * s_ref[...] + t_ref[...]
    o_ref[...] = _lrelu(_bdot(hb, w_ref[...]) + b_ref[...])


def _mm_affine(x, s, t, w, b):
    n, cin = x.shape
    cout = w.shape[1]
    return pl.pallas_call(
        _mm_affine_body,
        grid=(n // BR,),
        in_specs=[
            pl.BlockSpec((BR, cin), lambda i: (i, 0)),
            pl.BlockSpec((1, cin), lambda i: (0, 0)),
            pl.BlockSpec((1, cin), lambda i: (0, 0)),
            pl.BlockSpec((cin, cout), lambda i: (0, 0)),
            pl.BlockSpec((1, cout), lambda i: (0, 0)),
        ],
        out_specs=pl.BlockSpec((BR, cout), lambda i: (i, 0)),
        out_shape=jax.ShapeDtypeStruct((n, cout), jnp.float32),
    )(x, s, t, w, b)


def _dec_body(cur_ref, last_ref, mw_ref, mb_ref, dw1_ref, dw2_ref, db_ref,
              f_ref, st_ref):
    skip = _lrelu(_bdot(cur_ref[...], mw_ref[...]) + mb_ref[...])
    f = (_bdot(skip, dw1_ref[...]) + _bdot(last_ref[...], dw2_ref[...])
         + db_ref[...])
    f_ref[...] = f

    @pl.when(pl.program_id(0) == 0)
    def _():
        st_ref[...] = jnp.zeros_like(st_ref)

    st_ref[...] += jnp.stack([jnp.sum(f, axis=0), jnp.sum(f * f, axis=0)])


def _dec_mm(cur, last, mw, mb, dw1, dw2, db):
    n = cur.shape[0]
    c = 64
    return pl.pallas_call(
        _dec_body,
        grid=(n // BR,),
        in_specs=[
            pl.BlockSpec((BR, c), lambda i: (i, 0)),
            pl.BlockSpec((BR, c), lambda i: (i, 0)),
            pl.BlockSpec((c, c), lambda i: (0, 0)),
            pl.BlockSpec((1, c), lambda i: (0, 0)),
            pl.BlockSpec((c, c), lambda i: (0, 0)),
            pl.BlockSpec((c, c), lambda i: (0, 0)),
            pl.BlockSpec((1, c), lambda i: (0, 0)),
        ],
        out_specs=[
            pl.BlockSpec((BR, c), lambda i: (i, 0)),
            pl.BlockSpec((2, c), lambda i: (0, 0)),
        ],
        out_shape=[
            jax.ShapeDtypeStruct((n, c), jnp.float32),
            jax.ShapeDtypeStruct((2, c), jnp.float32),
        ],
    )(cur, last, mw, mb, dw1, dw2, db)


def _affine_body(f_ref, s_ref, t_ref, o_ref):
    o_ref[...] = _lrelu(f_ref[...] * s_ref[...] + t_ref[...])


def _affine(f, s, t):
    n, c = f.shape
    return pl.pallas_call(
        _affine_body,
        grid=(n // BR,),
        in_specs=[
            pl.BlockSpec((BR, c), lambda i: (i, 0)),
            pl.BlockSpec((1, c), lambda i: (0, 0)),
            pl.BlockSpec((1, c), lambda i: (0, 0)),
        ],
        out_specs=pl.BlockSpec((BR, c), lambda i: (i, 0)),
        out_shape=jax.ShapeDtypeStruct((n, c), jnp.float32),
    )(f, s, t)


def _bn_scale_shift(st, n, g, be):
    mean = st[0] / n
    var = st[1] / n - mean * mean
    s = g * lax.rsqrt(var + 1e-5)
    t = be - mean * s
    return s, t


_NW = 32
_CB = 2000
_GS = 1000


def _gather(table, idx):
    b = idx.shape[0]
    v = table.shape[0]
    nblocks = b // _CB

    @functools.partial(
        pl.kernel,
        mesh=plsc.VectorSubcoreMesh(core_axis_name="c", subcore_axis_name="s"),
        compiler_params=pltpu.CompilerParams(use_tc_tiling_on_sc=False),
        out_type=jax.ShapeDtypeStruct((b, 64), jnp.float32),
        scratch_types=[
            pltpu.VMEM((_CB,), jnp.int32),
            pltpu.VMEM((_GS, 64), jnp.float32),
            pltpu.SemaphoreType.DMA,
        ],
    )
    def k(table_hbm, idx_hbm, out_hbm, idx_v, rows_v, sem):
        wid = lax.axis_index("s") * 2 + lax.axis_index("c")

        def block_body(t, _):
            j = wid + t * _NW
            base = pl.multiple_of(j * _CB, 8)
            pltpu.sync_copy(idx_hbm.at[pl.ds(base, _CB)], idx_v)

            def sub(s, __):
                pltpu.async_copy(
                    table_hbm.at[idx_v.at[pl.ds(pl.multiple_of(s * _GS, 8), _GS)]], rows_v,
                    sem).wait()
                pltpu.sync_copy(rows_v,
                                out_hbm.at[pl.ds(base + s * _GS, _GS)])
                return __

            return lax.fori_loop(0, _CB // _GS, sub, _)

        nj = (nblocks - wid + _NW - 1) // _NW
        lax.fori_loop(0, nj, block_body, 0)

    return k(table, idx)


def _segmax(x, idx, num):
    o = jax.ops.segment_max(x, idx, num_segments=num)
    return jnp.where(jnp.isfinite(o), o, 0.0)


def _enc(x, w1, b1, g, be, w2, b2):
    n = x.shape[0]
    h, st = _mm_stats(x, w1, b1.reshape(1, -1))
    s, t = _bn_scale_shift(st, n, g, be)
    return _mm_affine(h, s.reshape(1, -1), t.reshape(1, -1), w2,
                      b2.reshape(1, -1))


def _ur(cur, last, mw, mb, dw, db, g, be):
    n = cur.shape[0]
    f, st = _dec_mm(cur, last, mw, mb.reshape(1, -1), dw[:64], dw[64:],
                    db.reshape(1, -1))
    s, t = _bn_scale_shift(st, n, g, be)
    return _affine(f, s.reshape(1, -1), t.reshape(1, -1))


def kernel(pt_fea, coors_inv_2, coors_inv_4, coors_inv_8, coors_inv_16,
           enc0_W1, encW1, encB1, encG, encBe, encW2, encB2,
           mlpW, mlpB, decW, decB, decG, decBe):
    p1 = _enc(pt_fea, enc0_W1, encB1[0], encG[0], encBe[0], encW2[0], encB2[0])
    o = _segmax(p1, coors_inv_2, N2)
    p2 = _enc(o, encW1[0], encB1[1], encG[1], encBe[1], encW2[1], encB2[1])
    o = _segmax(p2, coors_inv_4, N4)
    p3 = _enc(o, encW1[1], encB1[2], encG[2], encBe[2], encW2[2], encB2[2])
    o = _segmax(p3, coors_inv_8, N8)
    p4 = _enc(o, encW1[2], encB1[3], encG[3], encBe[3], encW2[3], encB2[3])
    o = _segmax(p4, coors_inv_16, N16)
    p5 = _enc(o, encW1[3], encB1[4], encG[4], encBe[4], encW2[4], encB2[4])
    p5f = _ur(p5, p5, mlpW[0], mlpB[0], decW[0], decB[0], decG[0], decBe[0])
    o = _gather(p5f, coors_inv_16)
    p4f = _ur(p4, o, mlpW[1], mlpB[1], decW[1], decB[1], decG[1], decBe[1])
    o = _gather(p4f, coors_inv_8)
    p3f = _ur(p3, o, mlpW[2], mlpB[2], decW[2], decB[2], decG[2], decBe[2])
    o = _gather(p3f, coors_inv_4)
    p2f = _ur(p2, o, mlpW[3], mlpB[3], decW[3], decB[3], decG[3], decBe[3])
    o = _gather(p2f, coors_inv_2)
    p1f = _ur(p1, o, mlpW[4], mlpB[4], decW[4], decB[4], decG[4], decBe[4])
    return (p1f, p2f, p3f, p4f, p5f)


# speedup vs baseline: 1.5529x; 1.1466x over previous
import functools

import jax
import jax.numpy as jnp
from jax import lax
from jax.experimental import pallas as pl
from jax.experimental.pallas import tpu as pltpu
from jax.experimental.pallas import tpu_sc as plsc

N1, N2, N4, N8, N16 = 500000, 200000, 80000, 30000, 10000
BR = 10000


def _lrelu(x):
    return jnp.where(x >= 0, x, 0.1 * x)


def _bdot(a, b):
    return jnp.dot(a.astype(jnp.bfloat16), b.astype(jnp.bfloat16),
                   preferred_element_type=jnp.float32)


def _mm_stats_body(x_ref, w_ref, b_ref, h_ref, st_ref):
    h = _lrelu(_bdot(x_ref[...], w_ref[...]) + b_ref[...])
    h_ref[...] = h

    @pl.when(pl.program_id(0) == 0)
    def _():
        st_ref[...] = jnp.zeros_like(st_ref)

    st_ref[...] += jnp.stack([jnp.sum(h, axis=0), jnp.sum(h * h, axis=0)])


def _mm_stats(x, w, b):
    n, cin = x.shape
    cout = w.shape[1]
    grid = n // BR
    return pl.pallas_call(
        _mm_stats_body,
        grid=(grid,),
        in_specs=[
            pl.BlockSpec((BR, cin), lambda i: (i, 0)),
            pl.BlockSpec((cin, cout), lambda i: (0, 0)),
            pl.BlockSpec((1, cout), lambda i: (0, 0)),
        ],
        out_specs=[
            pl.BlockSpec((BR, cout), lambda i: (i, 0)),
            pl.BlockSpec((2, cout), lambda i: (0, 0)),
        ],
        out_shape=[
            jax.ShapeDtypeStruct((n, cout), jnp.float32),
            jax.ShapeDtypeStruct((2, cout), jnp.float32),
        ],
    )(x, w, b)


def _mm_affine_body(x_ref, s_ref, t_ref, w_ref, b_ref, o_ref):
    hb = x_ref[...] * s_ref[...] + t_ref[...]
    o_ref[...] = _lrelu(_bdot(hb, w_ref[...]) + b_ref[...])


def _mm_affine(x, s, t, w, b):
    n, cin = x.shape
    cout = w.shape[1]
    return pl.pallas_call(
        _mm_affine_body,
        grid=(n // BR,),
        in_specs=[
            pl.BlockSpec((BR, cin), lambda i: (i, 0)),
            pl.BlockSpec((1, cin), lambda i: (0, 0)),
            pl.BlockSpec((1, cin), lambda i: (0, 0)),
            pl.BlockSpec((cin, cout), lambda i: (0, 0)),
            pl.BlockSpec((1, cout), lambda i: (0, 0)),
        ],
        out_specs=pl.BlockSpec((BR, cout), lambda i: (i, 0)),
        out_shape=jax.ShapeDtypeStruct((n, cout), jnp.float32),
    )(x, s, t, w, b)


def _dec_body(cur_ref, last_ref, mw_ref, mb_ref, dw1_ref, dw2_ref, db_ref,
              f_ref, st_ref):
    skip = _lrelu(_bdot(cur_ref[...], mw_ref[...]) + mb_ref[...])
    f = (_bdot(skip, dw1_ref[...]) + _bdot(last_ref[...], dw2_ref[...])
         + db_ref[...])
    f_ref[...] = f

    @pl.when(pl.program_id(0) == 0)
    def _():
        st_ref[...] = jnp.zeros_like(st_ref)

    st_ref[...] += jnp.stack([jnp.sum(f, axis=0), jnp.sum(f * f, axis=0)])


def _dec_mm(cur, last, mw, mb, dw1, dw2, db):
    n = cur.shape[0]
    c = 64
    return pl.pallas_call(
        _dec_body,
        grid=(n // BR,),
        in_specs=[
            pl.BlockSpec((BR, c), lambda i: (i, 0)),
            pl.BlockSpec((BR, c), lambda i: (i, 0)),
            pl.BlockSpec((c, c), lambda i: (0, 0)),
            pl.BlockSpec((1, c), lambda i: (0, 0)),
            pl.BlockSpec((c, c), lambda i: (0, 0)),
            pl.BlockSpec((c, c), lambda i: (0, 0)),
            pl.BlockSpec((1, c), lambda i: (0, 0)),
        ],
        out_specs=[
            pl.BlockSpec((BR, c), lambda i: (i, 0)),
            pl.BlockSpec((2, c), lambda i: (0, 0)),
        ],
        out_shape=[
            jax.ShapeDtypeStruct((n, c), jnp.float32),
            jax.ShapeDtypeStruct((2, c), jnp.float32),
        ],
    )(cur, last, mw, mb, dw1, dw2, db)


def _affine_body(f_ref, s_ref, t_ref, o_ref):
    o_ref[...] = _lrelu(f_ref[...] * s_ref[...] + t_ref[...])


def _affine(f, s, t):
    n, c = f.shape
    return pl.pallas_call(
        _affine_body,
        grid=(n // BR,),
        in_specs=[
            pl.BlockSpec((BR, c), lambda i: (i, 0)),
            pl.BlockSpec((1, c), lambda i: (0, 0)),
            pl.BlockSpec((1, c), lambda i: (0, 0)),
        ],
        out_specs=pl.BlockSpec((BR, c), lambda i: (i, 0)),
        out_shape=jax.ShapeDtypeStruct((n, c), jnp.float32),
    )(f, s, t)


def _bn_scale_shift(st, n, g, be):
    mean = st[0] / n
    var = st[1] / n - mean * mean
    s = g * lax.rsqrt(var + 1e-5)
    t = be - mean * s
    return s, t


_NW = 32
_CB = 2000
_GS = 1000


def _gather(table, idx):
    b = idx.shape[0]
    v = table.shape[0]
    nblocks = b // _CB

    @functools.partial(
        pl.kernel,
        mesh=plsc.VectorSubcoreMesh(core_axis_name="c", subcore_axis_name="s"),
        compiler_params=pltpu.CompilerParams(use_tc_tiling_on_sc=False),
        out_type=jax.ShapeDtypeStruct((b, 64), jnp.float32),
        scratch_types=[
            pltpu.VMEM((_CB,), jnp.int32),
            pltpu.VMEM((_GS, 64), jnp.float32),
            pltpu.SemaphoreType.DMA,
        ],
    )
    def k(table_hbm, idx_hbm, out_hbm, idx_v, rows_v, sem):
        wid = lax.axis_index("s") * 2 + lax.axis_index("c")

        def block_body(t, _):
            j = wid + t * _NW
            base = pl.multiple_of(j * _CB, 8)
            pltpu.sync_copy(idx_hbm.at[pl.ds(base, _CB)], idx_v)

            def sub(s, __):
                pltpu.async_copy(
                    table_hbm.at[idx_v.at[pl.ds(pl.multiple_of(s * _GS, 8), _GS)]], rows_v,
                    sem).wait()
                pltpu.sync_copy(rows_v,
                                out_hbm.at[pl.ds(base + s * _GS, _GS)])
                return __

            return lax.fori_loop(0, _CB // _GS, sub, _)

        nj = (nblocks - wid + _NW - 1) // _NW
        lax.fori_loop(0, nj, block_body, 0)

    return k(table, idx)


def _segmax(x, idx, num):
    o = jax.ops.segment_max(x, idx, num_segments=num)
    return jnp.where(jnp.isfinite(o), o, 0.0)


def _enc(x, w1, b1, g, be, w2, b2):
    n = x.shape[0]
    h, st = _mm_stats(x, w1, b1.reshape(1, -1))
    s, t = _bn_scale_shift(st, n, g, be)
    return _mm_affine(h, s.reshape(1, -1), t.reshape(1, -1), w2,
                      b2.reshape(1, -1))


def _ur(cur, last, mw, mb, dw, db, g, be):
    n = cur.shape[0]
    f, st = _dec_mm(cur, last, mw, mb.reshape(1, -1), dw[:64], dw[64:],
                    db.reshape(1, -1))
    s, t = _bn_scale_shift(st, n, g, be)
    return _affine(f, s.reshape(1, -1), t.reshape(1, -1))


def kernel(pt_fea, coors_inv_2, coors_inv_4, coors_inv_8, coors_inv_16,
           enc0_W1, encW1, encB1, encG, encBe, encW2, encB2,
           mlpW, mlpB, decW, decB, decG, decBe):
    p1 = _enc(pt_fea, enc0_W1, encB1[0], encG[0], encBe[0], encW2[0], encB2[0])
    o = _segmax(p1, coors_inv_2, N2)
    p2 = _enc(o, encW1[0], encB1[1], encG[1], encBe[1], encW2[1], encB2[1])
    o = _segmax(p2, coors_inv_4, N4)
    p3 = _enc(o, encW1[1], encB1[2], encG[2], encBe[2], encW2[2], encB2[2])
    o = _segmax(p3, coors_inv_8, N8)
    p4 = _enc(o, encW1[2], encB1[3], encG[3], encBe[3], encW2[3], encB2[3])
    o = _segmax(p4, coors_inv_16, N16)
    p5 = _enc(o, encW1[3], encB1[4], encG[4], encBe[4], encW2[4], encB2[4])
    p5f = _ur(p5, p5, mlpW[0], mlpB[0], decW[0], decB[0], decG[0], decBe[0])
    o = _gather(p5f, coors_inv_16)
    p4f = _ur(p4, o, mlpW[1], mlpB[1], decW[1], decB[1], decG[1], decBe[1])
    o = _gather(p4f, coors_inv_8)
    p3f = _ur(p3, o, mlpW[2], mlpB[2], decW[2], decB[2], decG[2], decBe[2])
    o = _gather(p3f, coors_inv_4)
    p2f = _ur(p2, o, mlpW[3], mlpB[3], decW[3], decB[3], decG[3], decBe[3])
    o = _gather(p2f, coors_inv_2)
    p1f = _ur(p1, o, mlpW[4], mlpB[4], decW[4], decB[4], decG[4], decBe[4])
    return (p1f, p2f, p3f, p4f, p5f)
